# baseline (speedup 1.0000x reference)
import jax
import jax.numpy as jnp
from jax.experimental import pallas as pl

_TOTAL = 16 * 96 * 224 * 224
_LANES = 1024
_ROWS = _TOTAL // _LANES
_BLOCK_ROWS = 512
_GRID = _ROWS // _BLOCK_ROWS


def _loss_kernel(out_ref, tgt_ref, acc_ref):
    i = pl.program_id(0)
    o = out_ref[...]
    t = tgt_ref[...]
    d = o - t
    ad = jnp.abs(d)
    elem = jnp.where(ad < 1.0, 0.5 * d * d, ad - 0.5)
    elem = jnp.where(t != 0.0, elem, 0.0)
    part = jnp.sum(elem)[None, None]

    @pl.when(i == 0)
    def _init():
        acc_ref[...] = part

    @pl.when(i > 0)
    def _acc():
        acc_ref[...] = acc_ref[...] + part


def kernel(out, target):
    o2 = out.reshape(_ROWS, _LANES)
    t2 = target.reshape(_ROWS, _LANES)
    res = pl.pallas_call(
        _loss_kernel,
        grid=(_GRID,),
        in_specs=[
            pl.BlockSpec((_BLOCK_ROWS, _LANES), lambda i: (i, 0)),
            pl.BlockSpec((_BLOCK_ROWS, _LANES), lambda i: (i, 0)),
        ],
        out_specs=pl.BlockSpec((1, 1), lambda i: (0, 0)),
        out_shape=jax.ShapeDtypeStruct((1, 1), jnp.float32),
    )(o2, t2)
    return res[0, 0]


# speedup vs baseline: 1.0241x; 1.0241x over previous
import jax
import jax.numpy as jnp
from jax.experimental import pallas as pl
from jax.experimental.pallas import tpu as pltpu

_TOTAL = 16 * 96 * 224 * 224
_LANES = 1024
_ROWS = _TOTAL // _LANES
_BLOCK_ROWS = 1536
_GRID = _ROWS // _BLOCK_ROWS


def _loss_kernel(out_ref, tgt_ref, res_ref, acc_ref):
    i = pl.program_id(0)
    o = out_ref[...]
    t = tgt_ref[...]
    d = o - t
    a = jnp.abs(d)
    m = jnp.minimum(a, 1.0)
    f = m * (a - 0.5 * m)
    f = jnp.where(t != 0.0, f, 0.0)
    part = jnp.sum(f.reshape(-1, 8, _LANES), axis=0)

    @pl.when(i == 0)
    def _init():
        acc_ref[...] = part

    @pl.when(i > 0)
    def _acc():
        acc_ref[...] = acc_ref[...] + part

    @pl.when(i == _GRID - 1)
    def _fin():
        res_ref[...] = jnp.sum(acc_ref[...])[None, None]


def kernel(out, target):
    o2 = out.reshape(_ROWS, _LANES)
    t2 = target.reshape(_ROWS, _LANES)
    res = pl.pallas_call(
        _loss_kernel,
        grid=(_GRID,),
        in_specs=[
            pl.BlockSpec((_BLOCK_ROWS, _LANES), lambda i: (i, 0)),
            pl.BlockSpec((_BLOCK_ROWS, _LANES), lambda i: (i, 0)),
        ],
        out_specs=pl.BlockSpec((1, 1), lambda i: (0, 0)),
        out_shape=jax.ShapeDtypeStruct((1, 1), jnp.float32),
        scratch_shapes=[pltpu.VMEM((8, _LANES), jnp.float32)],
    )(o2, t2)
    return res[0, 0]


# speedup vs baseline: 1.0586x; 1.0337x over previous
import jax
import jax.numpy as jnp
from jax.experimental import pallas as pl
from jax.experimental.pallas import tpu as pltpu

_TOTAL = 16 * 96 * 224 * 224
_LANES = 1024
_ROWS = _TOTAL // _LANES
_S = 8
_SEG = _ROWS // _S
_B = 192
_GRID = _SEG // _B


def _loss_kernel(*refs):
    out_refs = refs[:_S]
    tgt_refs = refs[_S:2 * _S]
    res_ref = refs[2 * _S]
    acc_ref = refs[2 * _S + 1]
    i = pl.program_id(0)

    part = None
    for k in range(_S):
        o = out_refs[k][...]
        t = tgt_refs[k][...]
        d = o - t
        a = jnp.abs(d)
        m = jnp.minimum(a, 1.0)
        f = m * (a - 0.5 * m)
        f = jnp.where(t != 0.0, f, 0.0)
        p = jnp.sum(f.reshape(-1, 8, _LANES), axis=0)
        part = p if part is None else part + p

    @pl.when(i == 0)
    def _init():
        acc_ref[...] = part

    @pl.when(i > 0)
    def _acc():
        acc_ref[...] = acc_ref[...] + part

    @pl.when(i == _GRID - 1)
    def _fin():
        res_ref[...] = jnp.sum(acc_ref[...])[None, None]


def _spec(k):
    return pl.BlockSpec((_B, _LANES), lambda i, k=k: (k * _GRID + i, 0))


def kernel(out, target):
    o2 = out.reshape(_ROWS, _LANES)
    t2 = target.reshape(_ROWS, _LANES)
    specs = [_spec(k) for k in range(_S)]
    res = pl.pallas_call(
        _loss_kernel,
        grid=(_GRID,),
        in_specs=specs + specs,
        out_specs=pl.BlockSpec((1, 1), lambda i: (0, 0)),
        out_shape=jax.ShapeDtypeStruct((1, 1), jnp.float32),
        scratch_shapes=[pltpu.VMEM((8, _LANES), jnp.float32)],
    )(*([o2] * _S + [t2] * _S))
    return res[0, 0]


# speedup vs baseline: 4.1738x; 3.9428x over previous
import jax
import jax.numpy as jnp
from jax.experimental import pallas as pl
from jax.experimental.pallas import tpu as pltpu

_W = 224
_ROWS = 16 * 96 * 224
_B = 3584
_GRID = _ROWS // _B


def _loss_kernel(out_ref, tgt_ref, res_ref, acc_ref):
    i = pl.program_id(0)
    o = out_ref[...]
    t = tgt_ref[...]
    d = o - t
    a = jnp.abs(d)
    m = jnp.minimum(a, 1.0)
    f = m * (a - 0.5 * m)
    f = jnp.where(t != 0.0, f, 0.0)
    part = jnp.sum(f.reshape(-1, 8, _W), axis=0)

    @pl.when(i == 0)
    def _init():
        acc_ref[...] = part

    @pl.when(i > 0)
    def _acc():
        acc_ref[...] = acc_ref[...] + part

    @pl.when(i == _GRID - 1)
    def _fin():
        res_ref[...] = jnp.sum(acc_ref[...])[None, None]


def kernel(out, target):
    o2 = out.reshape(_ROWS, _W)
    t2 = target.reshape(_ROWS, _W)
    res = pl.pallas_call(
        _loss_kernel,
        grid=(_GRID,),
        in_specs=[
            pl.BlockSpec((_B, _W), lambda i: (i, 0)),
            pl.BlockSpec((_B, _W), lambda i: (i, 0)),
        ],
        out_specs=pl.BlockSpec((1, 1), lambda i: (0, 0)),
        out_shape=jax.ShapeDtypeStruct((1, 1), jnp.float32),
        scratch_shapes=[pltpu.VMEM((8, _W), jnp.float32)],
    )(o2, t2)
    return res[0, 0]


# speedup vs baseline: 4.5143x; 1.0816x over previous
import jax
import jax.numpy as jnp
from jax.experimental import pallas as pl
from jax.experimental.pallas import tpu as pltpu

_W = 224
_ROWS = 16 * 96 * 224
_B = 7168
_GRID = _ROWS // _B


def _loss_kernel(out_ref, tgt_ref, res_ref, acc_ref):
    i = pl.program_id(0)
    o = out_ref[...]
    t = tgt_ref[...]
    d = o - t
    a = jnp.abs(d)
    m = jnp.minimum(a, 1.0)
    f = m * (a - 0.5 * m)
    f = jnp.where(t != 0.0, f, 0.0)
    part = jnp.sum(f.reshape(-1, 8, _W), axis=0)

    @pl.when(i == 0)
    def _init():
        acc_ref[...] = part

    @pl.when(i > 0)
    def _acc():
        acc_ref[...] = acc_ref[...] + part

    @pl.when(i == _GRID - 1)
    def _fin():
        res_ref[...] = jnp.sum(acc_ref[...])[None, None]


def kernel(out, target):
    o2 = out.reshape(_ROWS, _W)
    t2 = target.reshape(_ROWS, _W)
    res = pl.pallas_call(
        _loss_kernel,
        grid=(_GRID,),
        in_specs=[
            pl.BlockSpec((_B, _W), lambda i: (i, 0)),
            pl.BlockSpec((_B, _W), lambda i: (i, 0)),
        ],
        out_specs=pl.BlockSpec((1, 1), lambda i: (0, 0)),
        out_shape=jax.ShapeDtypeStruct((1, 1), jnp.float32),
        scratch_shapes=[pltpu.VMEM((8, _W), jnp.float32)],
    )(o2, t2)
    return res[0, 0]


# speedup vs baseline: 4.9398x; 1.0943x over previous
import jax
import jax.numpy as jnp
from jax.experimental import pallas as pl
from jax.experimental.pallas import tpu as pltpu

_W = 224
_ROWS = 16 * 96 * 224
_B = 7168
_GRID = _ROWS // _B


def _loss_kernel(out_ref, tgt_ref, res_ref):
    o = out_ref[...]
    t = tgt_ref[...]
    d = o - t
    a = jnp.abs(d)
    m = jnp.minimum(a, 1.0)
    f = m * (a - 0.5 * m)
    f = jnp.where(t != 0.0, f, 0.0)
    res_ref[...] = jnp.sum(f)[None, None, None]


def kernel(out, target):
    o2 = out.reshape(_ROWS, _W)
    t2 = target.reshape(_ROWS, _W)
    res = pl.pallas_call(
        _loss_kernel,
        grid=(_GRID,),
        in_specs=[
            pl.BlockSpec((_B, _W), lambda i: (i, 0)),
            pl.BlockSpec((_B, _W), lambda i: (i, 0)),
        ],
        out_specs=pl.BlockSpec((1, 1, 1), lambda i: (i, 0, 0)),
        out_shape=jax.ShapeDtypeStruct((_GRID, 1, 1), jnp.float32),
        compiler_params=pltpu.CompilerParams(
            dimension_semantics=("parallel",),
        ),
    )(o2, t2)
    return jnp.sum(res)
